# baseline (speedup 1.0000x reference)
import functools

import jax
import jax.numpy as jnp
from jax import lax
from jax.experimental import pallas as pl
from jax.experimental.pallas import tpu as pltpu
from jax.experimental.pallas import tpu_sc as plsc

B = 4096
F = 26
D = 32
NC = 2
NS = 16
NW = NC * NS
BPW = B // NW
L = 16


def _sc_body(emb_hbm, lin_hbm, idx_hbm, s_out, q_out, l_out,
             idx_v, ebuf, lbuf, acc_s, acc_q, acc_l, esem, lsem):
    c = lax.axis_index("c")
    s = lax.axis_index("s")
    wid = s * NC + c

    pltpu.sync_copy(idx_hbm.at[wid], idx_v)

    edescs = []
    ldescs = []
    for f in range(F):
        edescs.append(pltpu.async_copy(emb_hbm.at[idx_v.at[f]], ebuf.at[f], esem))
        ldescs.append(pltpu.async_copy(lin_hbm.at[idx_v.at[f]], lbuf.at[f], lsem))
    for d_ in edescs:
        d_.wait()

    def row_body(r, carry):
        for half in range(D // L):
            sl = pl.ds(half * L, L)
            vs = [ebuf[f, r, sl] for f in range(F)]
            qs = [v * v for v in vs]
            while len(vs) > 1:
                vs = [vs[i] + vs[i + 1] for i in range(0, len(vs) - 1, 2)] + (
                    [vs[-1]] if len(vs) % 2 else [])
                qs = [qs[i] + qs[i + 1] for i in range(0, len(qs) - 1, 2)] + (
                    [qs[-1]] if len(qs) % 2 else [])
            acc_s[r, sl] = vs[0]
            acc_q[r, sl] = qs[0]
        return carry

    lax.fori_loop(0, BPW, row_body, 0)

    for d_ in ldescs:
        d_.wait()
    for k in range(BPW // L):
        sl = pl.ds(k * L, L)
        vs = [lbuf[f, sl] for f in range(F)]
        while len(vs) > 1:
            vs = [vs[i] + vs[i + 1] for i in range(0, len(vs) - 1, 2)] + (
                [vs[-1]] if len(vs) % 2 else [])
        acc_l[sl] = vs[0]

    base = wid * BPW
    pltpu.sync_copy(acc_s, s_out.at[pl.ds(base, BPW)])
    pltpu.sync_copy(acc_q, q_out.at[pl.ds(base, BPW)])
    pltpu.sync_copy(acc_l, l_out.at[pl.ds(base, BPW)])


def _sc_reduce(emb_table, lin_flat, idx_arr):
    mesh = plsc.VectorSubcoreMesh(core_axis_name="c", subcore_axis_name="s")
    fn = functools.partial(
        pl.kernel,
        mesh=mesh,
        compiler_params=pltpu.CompilerParams(use_tc_tiling_on_sc=False),
        out_type=[
            jax.ShapeDtypeStruct((B, D), jnp.float32),
            jax.ShapeDtypeStruct((B, D), jnp.float32),
            jax.ShapeDtypeStruct((B,), jnp.float32),
        ],
        scratch_types=[
            pltpu.VMEM((F, BPW), jnp.int32),
            pltpu.VMEM((F, BPW, D), jnp.float32),
            pltpu.VMEM((F, BPW), jnp.float32),
            pltpu.VMEM((BPW, D), jnp.float32),
            pltpu.VMEM((BPW, D), jnp.float32),
            pltpu.VMEM((BPW,), jnp.float32),
            pltpu.SemaphoreType.DMA,
            pltpu.SemaphoreType.DMA,
        ],
    )(_sc_body)
    return fn(emb_table, lin_flat, idx_arr)


def _tc_body(s_ref, q_ref, l_ref, w1, b1, w2, b2, w3, b3, o_ref):
    sv = s_ref[...]
    qv = q_ref[...]
    bi = 0.5 * (sv * sv - qv)
    h = jnp.maximum(jnp.dot(bi, w1[...], preferred_element_type=jnp.float32)
                    + b1[...], 0.0)
    h = jnp.maximum(jnp.dot(h, w2[...], preferred_element_type=jnp.float32)
                    + b2[...], 0.0)
    deep = jnp.dot(h, w3[...], preferred_element_type=jnp.float32)
    o_ref[...] = l_ref[...] + deep + b3[...]


def _tc_mlp(S, Q, Lsum, W1, b1, W2, b2, W3, b3):
    out = pl.pallas_call(
        _tc_body,
        out_shape=jax.ShapeDtypeStruct((B, 1), jnp.float32),
    )(S, Q, Lsum.reshape(B, 1), W1, b1[None], W2, b2[None], W3, b3[None])
    return out.reshape(B)


def kernel(features, emb_table, lin_table, W1, b1, W2, b2, W3, b3):
    idx_arr = (features.astype(jnp.int32).T
               .reshape(F, NW, BPW).transpose(1, 0, 2))
    lin_flat = lin_table.reshape(-1)
    S, Q, Lsum = _sc_reduce(emb_table, lin_flat, idx_arr)
    return _tc_mlp(S, Q, Lsum, W1, b1, W2, b2, W3, b3)
